# Initial kernel scaffold; baseline (speedup 1.0000x reference)
import functools
import numpy as np
import jax
import jax.numpy as jnp
from jax import lax
from jax.experimental import pallas as pl
from jax.experimental.pallas import tpu as pltpu
from jax.experimental.pallas import tpu_sc as plsc

_N = 10000
_E = 320000


def _tc1_body(x_ref, w_ref, g_ref, gd_ref, src_ref, dst_ref):
    h = jnp.dot(x_ref[...], w_ref[...], preferred_element_type=jnp.float32)
    src_ref[...] = jnp.dot(h, g_ref[...], preferred_element_type=jnp.float32)
    dst_ref[...] = jnp.dot(h, gd_ref[...], preferred_element_type=jnp.float32)


def _tc2_body(a0_ref, a1_ref, s_ref, r_ref, b_ref, w2_ref, g2_ref, gd2_ref,
              src2_ref, dst2_ref):
    acc = a0_ref[...] + a1_ref[...]
    numer = jnp.dot(acc, s_ref[...], preferred_element_type=jnp.float32)
    denom = jnp.dot(acc, r_ref[...], preferred_element_type=jnp.float32)
    o = numer / (denom + 1e-16) + b_ref[...]
    o = jnp.where(o > 0, o, jnp.expm1(o))
    h2 = jnp.dot(o, w2_ref[...], preferred_element_type=jnp.float32)
    src2_ref[...] = jnp.dot(h2, g2_ref[...], preferred_element_type=jnp.float32)
    dst2_ref[...] = jnp.dot(h2, gd2_ref[...], preferred_element_type=jnp.float32)


def _tc3_body(a0_ref, a1_ref, s_ref, r_ref, b_ref, out_ref):
    acc = a0_ref[...] + a1_ref[...]
    numer = jnp.dot(acc, s_ref[...], preferred_element_type=jnp.float32)
    denom = jnp.dot(acc, r_ref[...], preferred_element_type=jnp.float32)
    z = numer / (denom + 1e-16) + b_ref[...]
    m = jnp.max(z, axis=1, keepdims=True)
    out_ref[...] = z - (m + jnp.log(jnp.sum(jnp.exp(z - m), axis=1,
                                            keepdims=True)))


def _make_sc_edge_kernel(n, e, dh, heads):
    row = dh + 16
    info = plsc.get_sparse_core_info()
    nc, ns = info.num_cores, info.num_subcores
    nw = nc * ns
    epw = e // nw
    B = 80
    nchunks = epw // B
    rpt = n // ns
    zr = 125
    assert epw * nw == e and nchunks * B == epw and rpt % zr == 0
    mesh = plsc.VectorSubcoreMesh(core_axis_name="c", subcore_axis_name="s")

    @functools.partial(
        pl.kernel,
        out_type=jax.ShapeDtypeStruct((nc, n, row), jnp.float32),
        mesh=mesh,
        scratch_types=[
            pltpu.VMEM((B,), jnp.int32),
            pltpu.VMEM((B,), jnp.int32),
            pltpu.VMEM((B, row), jnp.float32),
            pltpu.VMEM((B, 16), jnp.float32),
            pltpu.VMEM((B, row), jnp.float32),
            pltpu.VMEM((16,), jnp.float32),
            pltpu.VMEM((125, row), jnp.float32),
            pltpu.VMEM_SHARED((n, row), jnp.float32),
            pltpu.SemaphoreType.DMA,
            pltpu.SemaphoreType.DMA,
        ],
    )
    def k(src_hbm, dst_hbm, stab_hbm, dtab_hbm, out_hbm,
          sidx, didx, rows, drows, orows, pbuf, zbuf, accum, sem1, sem2):
        cid = lax.axis_index("c")
        sid = lax.axis_index("s")
        wid = sid * nc + cid
        zero = jnp.zeros((16,), jnp.float32)
        zr = 125

        def zrow(i, carry):
            for t in range(row // 16):
                zbuf[i, pl.ds(16 * t, 16)] = zero
            return carry

        lax.fori_loop(0, zr, zrow, 0)
        for t in range(rpt // zr):
            pltpu.sync_copy(zbuf, accum.at[pl.ds(sid * rpt + t * zr, zr)])
        plsc.subcore_barrier()

        def chunk(j, carry):
            base = wid * epw + j * B
            pltpu.sync_copy(src_hbm.at[pl.ds(base, B)], sidx)
            pltpu.sync_copy(dst_hbm.at[pl.ds(base, B)], didx)
            c1 = pltpu.async_copy(stab_hbm.at[sidx], rows, sem1)
            c2 = pltpu.async_copy(dtab_hbm.at[didx], drows, sem2)
            c1.wait()
            c2.wait()

            def edge(ei, ecarry):
                vas = rows[ei, pl.ds(dh, 16)]
                vad = drows[ei, pl.ds(0, 16)]
                a = vas + vad
                a = jnp.where(a >= 0, a, 0.2 * a)
                p = jnp.exp(a)
                pbuf[...] = p
                orows[ei, pl.ds(dh, 16)] = p
                for kk in range(dh // 16):
                    if heads == 1:
                        pidx = jnp.zeros((16,), jnp.int32)
                    else:
                        pidx = (2 * kk) + lax.shift_right_logical(
                            lax.iota(jnp.int32, 16), 3)
                    sv = plsc.load_gather(pbuf, [pidx])
                    orows[ei, pl.ds(16 * kk, 16)] = (
                        rows[ei, pl.ds(16 * kk, 16)] * sv)
                return ecarry

            lax.fori_loop(0, B, edge, 0)
            pltpu.sync_copy(orows, accum.at[didx], add=True)
            return carry

        lax.fori_loop(0, nchunks, chunk, 0)
        plsc.subcore_barrier()

        for t in range(rpt // zr):
            r0 = sid * rpt + t * zr
            pltpu.sync_copy(accum.at[pl.ds(r0, zr)], zbuf)
            pltpu.sync_copy(zbuf, out_hbm.at[cid, pl.ds(r0, zr)])

    return k


_sc_layer1 = _make_sc_edge_kernel(_N, _E, 64, 8)
_sc_layer2 = _make_sc_edge_kernel(_N, _E, 128, 1)


def _block_att(att, heads, ch):
    a = att.reshape(heads, ch)
    eye_h = jnp.eye(heads, dtype=jnp.float32)
    return (a[:, :, None] * eye_h[:, None, :]).reshape(heads * ch, heads)


def kernel(x, edge_index, W1, att_src1, att_dst1, b1, W2, att_src2,
           att_dst2, b2):
    f32 = jnp.float32
    src = edge_index[0]
    dst = edge_index[1]

    asrc1 = _block_att(att_src1, 8, 8)
    adst1 = _block_att(att_dst1, 8, 8)
    z64_8 = jnp.zeros((64, 8), f32)
    G1 = jnp.concatenate([jnp.eye(64, dtype=f32), asrc1, z64_8], axis=1)
    Gd1 = jnp.concatenate([adst1, z64_8], axis=1)

    as2 = att_src2.reshape(128, 1)
    ad2 = att_dst2.reshape(128, 1)
    z128_15 = jnp.zeros((128, 15), f32)
    G2 = jnp.concatenate([jnp.eye(128, dtype=f32), as2, z128_15], axis=1)
    Gd2 = jnp.concatenate([ad2, z128_15], axis=1)

    S1 = np.zeros((80, 64), np.float32)
    S1[:64, :64] = np.eye(64)
    R1 = np.zeros((80, 64), np.float32)
    for h in range(8):
        R1[64 + h, h * 8:(h + 1) * 8] = 1.0
    S2 = np.zeros((144, 128), np.float32)
    S2[:128, :128] = np.eye(128)
    R2 = np.zeros((144, 128), np.float32)
    R2[128, :] = 1.0
    S1, R1, S2, R2 = map(jnp.asarray, (S1, R1, S2, R2))

    b1r = b1.reshape(1, 64)
    b2r = b2.reshape(1, 128)

    RB = 1000
    G = _N // RB

    src_tab, dst_tab = pl.pallas_call(
        _tc1_body,
        grid=(G,),
        in_specs=[
            pl.BlockSpec((RB, 128), lambda i: (i, 0)),
            pl.BlockSpec((128, 64), lambda i: (0, 0)),
            pl.BlockSpec((64, 80), lambda i: (0, 0)),
            pl.BlockSpec((64, 16), lambda i: (0, 0)),
        ],
        out_specs=[
            pl.BlockSpec((RB, 80), lambda i: (i, 0)),
            pl.BlockSpec((RB, 16), lambda i: (i, 0)),
        ],
        out_shape=[
            jax.ShapeDtypeStruct((_N, 80), f32),
            jax.ShapeDtypeStruct((_N, 16), f32),
        ],
    )(x, W1, G1, Gd1)

    acc1 = _sc_layer1(src, dst, src_tab, dst_tab)

    src_tab2, dst_tab2 = pl.pallas_call(
        _tc2_body,
        grid=(G,),
        in_specs=[
            pl.BlockSpec((RB, 80), lambda i: (i, 0)),
            pl.BlockSpec((RB, 80), lambda i: (i, 0)),
            pl.BlockSpec((80, 64), lambda i: (0, 0)),
            pl.BlockSpec((80, 64), lambda i: (0, 0)),
            pl.BlockSpec((1, 64), lambda i: (0, 0)),
            pl.BlockSpec((64, 128), lambda i: (0, 0)),
            pl.BlockSpec((128, 144), lambda i: (0, 0)),
            pl.BlockSpec((128, 16), lambda i: (0, 0)),
        ],
        out_specs=[
            pl.BlockSpec((RB, 144), lambda i: (i, 0)),
            pl.BlockSpec((RB, 16), lambda i: (i, 0)),
        ],
        out_shape=[
            jax.ShapeDtypeStruct((_N, 144), f32),
            jax.ShapeDtypeStruct((_N, 16), f32),
        ],
    )(acc1[0], acc1[1], S1, R1, b1r, W2, G2, Gd2)

    acc2 = _sc_layer2(src, dst, src_tab2, dst_tab2)

    out = pl.pallas_call(
        _tc3_body,
        grid=(G,),
        in_specs=[
            pl.BlockSpec((RB, 144), lambda i: (i, 0)),
            pl.BlockSpec((RB, 144), lambda i: (i, 0)),
            pl.BlockSpec((144, 128), lambda i: (0, 0)),
            pl.BlockSpec((144, 128), lambda i: (0, 0)),
            pl.BlockSpec((1, 128), lambda i: (0, 0)),
        ],
        out_specs=pl.BlockSpec((RB, 128), lambda i: (i, 0)),
        out_shape=jax.ShapeDtypeStruct((_N, 128), f32),
    )(acc2[0], acc2[1], S2, R2, b2r)

    return out


# speedup vs baseline: 33.9345x; 33.9345x over previous
import functools
import numpy as np
import jax
import jax.numpy as jnp
from jax import lax
from jax.experimental import pallas as pl
from jax.experimental.pallas import tpu as pltpu
from jax.experimental.pallas import tpu_sc as plsc

_N = 10000
_E = 320000


def _tc1_body(x_ref, w_ref, g_ref, gd_ref, src_ref, dst_ref):
    h = jnp.dot(x_ref[...], w_ref[...], preferred_element_type=jnp.float32)
    src_ref[...] = jnp.dot(h, g_ref[...], preferred_element_type=jnp.float32)
    dst_ref[...] = jnp.dot(h, gd_ref[...], preferred_element_type=jnp.float32)


def _tc2_body(a0_ref, a1_ref, s_ref, r_ref, b_ref, w2_ref, g2_ref, gd2_ref,
              src2_ref, dst2_ref):
    acc = a0_ref[...] + a1_ref[...]
    numer = jnp.dot(acc, s_ref[...], preferred_element_type=jnp.float32)
    denom = jnp.dot(acc, r_ref[...], preferred_element_type=jnp.float32)
    o = numer / (denom + 1e-16) + b_ref[...]
    o = jnp.where(o > 0, o, jnp.exp(o) - 1.0)
    h2 = jnp.dot(o, w2_ref[...], preferred_element_type=jnp.float32)
    src2_ref[...] = jnp.dot(h2, g2_ref[...], preferred_element_type=jnp.float32)
    dst2_ref[...] = jnp.dot(h2, gd2_ref[...], preferred_element_type=jnp.float32)


def _tc3_body(a0_ref, a1_ref, s_ref, r_ref, b_ref, out_ref):
    acc = a0_ref[...] + a1_ref[...]
    numer = jnp.dot(acc, s_ref[...], preferred_element_type=jnp.float32)
    denom = jnp.dot(acc, r_ref[...], preferred_element_type=jnp.float32)
    z = numer / (denom + 1e-16) + b_ref[...]
    m = jnp.max(z, axis=1, keepdims=True)
    out_ref[...] = z - (m + jnp.log(jnp.sum(jnp.exp(z - m), axis=1,
                                            keepdims=True)))


def _make_sc_edge_kernel(n, e, dh, drep):
    row = dh + drep
    info = plsc.get_sparse_core_info()
    nc, ns = info.num_cores, info.num_subcores
    nw = nc * ns
    epw = e // nw
    B = 80
    nchunks = epw // B
    CH = 80
    nch = n // CH
    cpt = nch // ns
    rem = nch - cpt * ns
    assert epw * nw == e and nchunks * B == epw and nch * CH == n
    mesh = plsc.VectorSubcoreMesh(core_axis_name="c", subcore_axis_name="s")

    @functools.partial(
        pl.kernel,
        out_type=jax.ShapeDtypeStruct((nc, n, row), jnp.float32),
        mesh=mesh,
        compiler_params=pltpu.CompilerParams(use_tc_tiling_on_sc=False),
        scratch_types=[
            pltpu.VMEM((B,), jnp.int32),
            pltpu.VMEM((B,), jnp.int32),
            pltpu.VMEM((B, row), jnp.float32),
            pltpu.VMEM((B, drep), jnp.float32),
            pltpu.VMEM((B, row), jnp.float32),
            pltpu.VMEM((CH, row), jnp.float32),
            pltpu.VMEM_SHARED((n, row), jnp.float32),
            pltpu.SemaphoreType.DMA,
            pltpu.SemaphoreType.DMA,
        ],
    )
    def k(src_hbm, dst_hbm, stab_hbm, dtab_hbm, out_hbm,
          sidx, didx, rows, drows, orows, zbuf, accum, sem1, sem2):
        cid = lax.axis_index("c")
        sid = lax.axis_index("s")
        wid = sid * nc + cid
        zero = jnp.zeros((16,), jnp.float32)

        def zrow(i, carry):
            for t in range(row // 16):
                zbuf[i, pl.ds(16 * t, 16)] = zero
            return carry

        lax.fori_loop(0, CH, zrow, 0)
        for t in range(cpt):
            c0 = sid * cpt + t
            pltpu.sync_copy(zbuf, accum.at[pl.ds(c0 * CH, CH)])
        if rem:
            @pl.when(sid < rem)
            def _zero_extra():
                c0 = cpt * ns + sid
                pltpu.sync_copy(zbuf, accum.at[pl.ds(c0 * CH, CH)])
        plsc.subcore_barrier()

        def chunk(j, carry):
            base = wid * epw + j * B
            pltpu.sync_copy(src_hbm.at[pl.ds(base, B)], sidx)
            pltpu.sync_copy(dst_hbm.at[pl.ds(base, B)], didx)
            c1 = pltpu.async_copy(stab_hbm.at[sidx], rows, sem1)
            c2 = pltpu.async_copy(dtab_hbm.at[didx], drows, sem2)
            c1.wait()
            c2.wait()

            def edge(ei, ecarry):
                ps = []
                for kr in range(drep // 16):
                    a = rows[ei, pl.ds(dh + 16 * kr, 16)] + \
                        drows[ei, pl.ds(16 * kr, 16)]
                    a = jnp.where(a >= 0, a, 0.2 * a)
                    p = jnp.exp(a)
                    ps.append(p)
                    orows[ei, pl.ds(dh + 16 * kr, 16)] = p
                for kk in range(dh // 16):
                    orows[ei, pl.ds(16 * kk, 16)] = (
                        rows[ei, pl.ds(16 * kk, 16)] * ps[kk % len(ps)])
                return ecarry

            lax.fori_loop(0, B, edge, 0)
            pltpu.sync_copy(orows, accum.at[didx], add=True)
            return carry

        lax.fori_loop(0, nchunks, chunk, 0)
        plsc.subcore_barrier()

        def dump(c0):
            pltpu.sync_copy(accum.at[pl.ds(c0 * CH, CH)], zbuf)
            pltpu.sync_copy(zbuf, out_hbm.at[cid, pl.ds(c0 * CH, CH)])

        for t in range(cpt):
            dump(sid * cpt + t)
        if rem:
            @pl.when(sid < rem)
            def _dump_extra():
                dump(cpt * ns + sid)

    return k


_sc_layer1 = _make_sc_edge_kernel(_N, _E, 64, 64)
_sc_layer2 = _make_sc_edge_kernel(_N, _E, 128, 16)


def _block_att_rep(att, heads, ch):
    a = att.reshape(heads, ch)
    eye_h = jnp.eye(heads, dtype=jnp.float32)
    m = a[:, :, None, None] * eye_h[:, None, :, None]
    m = jnp.broadcast_to(m, (heads, ch, heads, ch))
    return m.reshape(heads * ch, heads * ch)


def kernel(x, edge_index, W1, att_src1, att_dst1, b1, W2, att_src2,
           att_dst2, b2):
    f32 = jnp.float32
    src = edge_index[0]
    dst = edge_index[1]

    asrc1 = _block_att_rep(att_src1, 8, 8)
    adst1 = _block_att_rep(att_dst1, 8, 8)
    G1 = jnp.concatenate([jnp.eye(64, dtype=f32), asrc1], axis=1)
    Gd1 = adst1

    as2 = jnp.tile(att_src2.reshape(128, 1), (1, 16))
    ad2 = jnp.tile(att_dst2.reshape(128, 1), (1, 16))
    G2 = jnp.concatenate([jnp.eye(128, dtype=f32), as2], axis=1)
    Gd2 = ad2

    S1 = np.zeros((128, 64), np.float32)
    S1[:64, :64] = np.eye(64)
    R1 = np.zeros((128, 64), np.float32)
    R1[64:, :64] = np.eye(64)
    S2 = np.zeros((144, 128), np.float32)
    S2[:128, :128] = np.eye(128)
    R2 = np.zeros((144, 128), np.float32)
    R2[128, :] = 1.0
    S1, R1, S2, R2 = map(jnp.asarray, (S1, R1, S2, R2))

    b1r = b1.reshape(1, 64)
    b2r = b2.reshape(1, 128)

    RB = 1000
    G = _N // RB

    src_tab, dst_tab = pl.pallas_call(
        _tc1_body,
        grid=(G,),
        in_specs=[
            pl.BlockSpec((RB, 128), lambda i: (i, 0)),
            pl.BlockSpec((128, 64), lambda i: (0, 0)),
            pl.BlockSpec((64, 128), lambda i: (0, 0)),
            pl.BlockSpec((64, 64), lambda i: (0, 0)),
        ],
        out_specs=[
            pl.BlockSpec((RB, 128), lambda i: (i, 0)),
            pl.BlockSpec((RB, 64), lambda i: (i, 0)),
        ],
        out_shape=[
            jax.ShapeDtypeStruct((_N, 128), f32),
            jax.ShapeDtypeStruct((_N, 64), f32),
        ],
    )(x, W1, G1, Gd1)

    acc1 = _sc_layer1(src, dst, src_tab, dst_tab)

    src_tab2, dst_tab2 = pl.pallas_call(
        _tc2_body,
        grid=(G,),
        in_specs=[
            pl.BlockSpec((RB, 128), lambda i: (i, 0)),
            pl.BlockSpec((RB, 128), lambda i: (i, 0)),
            pl.BlockSpec((128, 64), lambda i: (0, 0)),
            pl.BlockSpec((128, 64), lambda i: (0, 0)),
            pl.BlockSpec((1, 64), lambda i: (0, 0)),
            pl.BlockSpec((64, 128), lambda i: (0, 0)),
            pl.BlockSpec((128, 144), lambda i: (0, 0)),
            pl.BlockSpec((128, 16), lambda i: (0, 0)),
        ],
        out_specs=[
            pl.BlockSpec((RB, 144), lambda i: (i, 0)),
            pl.BlockSpec((RB, 16), lambda i: (i, 0)),
        ],
        out_shape=[
            jax.ShapeDtypeStruct((_N, 144), f32),
            jax.ShapeDtypeStruct((_N, 16), f32),
        ],
    )(acc1[0], acc1[1], S1, R1, b1r, W2, G2, Gd2)

    acc2 = _sc_layer2(src, dst, src_tab2, dst_tab2)

    out = pl.pallas_call(
        _tc3_body,
        grid=(G,),
        in_specs=[
            pl.BlockSpec((RB, 144), lambda i: (i, 0)),
            pl.BlockSpec((RB, 144), lambda i: (i, 0)),
            pl.BlockSpec((144, 128), lambda i: (0, 0)),
            pl.BlockSpec((144, 128), lambda i: (0, 0)),
            pl.BlockSpec((1, 128), lambda i: (0, 0)),
        ],
        out_specs=pl.BlockSpec((RB, 128), lambda i: (i, 0)),
        out_shape=jax.ShapeDtypeStruct((_N, 128), f32),
    )(acc2[0], acc2[1], S2, R2, b2r)

    return out


# speedup vs baseline: 58.2747x; 1.7173x over previous
import functools
import numpy as np
import jax
import jax.numpy as jnp
from jax import lax
from jax.experimental import pallas as pl
from jax.experimental.pallas import tpu as pltpu
from jax.experimental.pallas import tpu_sc as plsc

_N = 10000
_E = 320000


def _tc1_body(x_ref, w_ref, g_ref, gd_ref, src_ref, dst_ref):
    h = jnp.dot(x_ref[...], w_ref[...], preferred_element_type=jnp.float32)
    src_ref[...] = jnp.dot(h, g_ref[...], preferred_element_type=jnp.float32)
    dst_ref[...] = jnp.dot(h, gd_ref[...], preferred_element_type=jnp.float32)


def _tc2_body(a0_ref, a1_ref, s_ref, r_ref, b_ref, w2_ref, g2_ref, gd2_ref,
              src2_ref, dst2_ref):
    acc = a0_ref[...] + a1_ref[...]
    numer = jnp.dot(acc, s_ref[...], preferred_element_type=jnp.float32)
    denom = jnp.dot(acc, r_ref[...], preferred_element_type=jnp.float32)
    o = numer / (denom + 1e-16) + b_ref[...]
    o = jnp.where(o > 0, o, jnp.exp(o) - 1.0)
    h2 = jnp.dot(o, w2_ref[...], preferred_element_type=jnp.float32)
    src2_ref[...] = jnp.dot(h2, g2_ref[...], preferred_element_type=jnp.float32)
    dst2_ref[...] = jnp.dot(h2, gd2_ref[...], preferred_element_type=jnp.float32)


def _tc3_body(a0_ref, a1_ref, s_ref, r_ref, b_ref, out_ref):
    acc = a0_ref[...] + a1_ref[...]
    numer = jnp.dot(acc, s_ref[...], preferred_element_type=jnp.float32)
    denom = jnp.dot(acc, r_ref[...], preferred_element_type=jnp.float32)
    z = numer / (denom + 1e-16) + b_ref[...]
    m = jnp.max(z, axis=1, keepdims=True)
    out_ref[...] = z - (m + jnp.log(jnp.sum(jnp.exp(z - m), axis=1,
                                            keepdims=True)))


def _make_sc_edge_kernel(n, e, dh, drep):
    row = dh + drep
    info = plsc.get_sparse_core_info()
    nc, ns = info.num_cores, info.num_subcores
    nw = nc * ns
    epw = e // nw
    B = 40
    nchunks = epw // B
    CH = 40
    nch = n // CH
    cpt = nch // ns
    rem = nch - cpt * ns
    assert epw * nw == e and nchunks * B == epw and nch * CH == n
    assert nchunks % 2 == 0
    mesh = plsc.VectorSubcoreMesh(core_axis_name="c", subcore_axis_name="s")

    @functools.partial(
        pl.kernel,
        out_type=jax.ShapeDtypeStruct((nc, n, row), jnp.float32),
        mesh=mesh,
        compiler_params=pltpu.CompilerParams(use_tc_tiling_on_sc=False),
        scratch_types=[
            pltpu.VMEM((B,), jnp.int32),
            pltpu.VMEM((B,), jnp.int32),
            pltpu.VMEM((B,), jnp.int32),
            pltpu.VMEM((B,), jnp.int32),
            pltpu.VMEM((B, row), jnp.float32),
            pltpu.VMEM((B, row), jnp.float32),
            pltpu.VMEM((B, drep), jnp.float32),
            pltpu.VMEM((B, drep), jnp.float32),
            pltpu.VMEM((B, row), jnp.float32),
            pltpu.VMEM((B, row), jnp.float32),
            pltpu.VMEM((CH, row), jnp.float32),
            pltpu.VMEM_SHARED((n, row), jnp.float32),
            pltpu.SemaphoreType.DMA,
            pltpu.SemaphoreType.DMA,
        ],
    )
    def k(src_hbm, dst_hbm, stab_hbm, dtab_hbm, out_hbm,
          sidx0, sidx1, didx0, didx1,
          rows0, rows1, drows0, drows1, orows0, orows1,
          zbuf, accum, gsem0, gsem1):
        sidx = (sidx0, sidx1)
        didx = (didx0, didx1)
        rows = (rows0, rows1)
        drows = (drows0, drows1)
        orows = (orows0, orows1)
        gsem = (gsem0, gsem1)
        cid = lax.axis_index("c")
        sid = lax.axis_index("s")
        wid = sid * nc + cid
        zero = jnp.zeros((16,), jnp.float32)

        def zrow(i, carry):
            for t in range(row // 16):
                zbuf[i, pl.ds(16 * t, 16)] = zero
            return carry

        lax.fori_loop(0, CH, zrow, 0)
        for t in range(cpt):
            c0 = sid * cpt + t
            pltpu.sync_copy(zbuf, accum.at[pl.ds(c0 * CH, CH)])
        if rem:
            @pl.when(sid < rem)
            def _zero_extra():
                c0 = cpt * ns + sid
                pltpu.sync_copy(zbuf, accum.at[pl.ds(c0 * CH, CH)])
        plsc.subcore_barrier()

        def fire(j, b):
            base = wid * epw + j * B
            pltpu.sync_copy(src_hbm.at[pl.ds(base, B)], sidx[b])
            pltpu.sync_copy(dst_hbm.at[pl.ds(base, B)], didx[b])
            pltpu.async_copy(stab_hbm.at[sidx[b]], rows[b], gsem[b])
            pltpu.async_copy(dtab_hbm.at[didx[b]], drows[b], gsem[b])

        def wait_gather(b):
            pltpu.make_async_copy(stab_hbm.at[sidx[b]], rows[b],
                                  gsem[b]).wait()
            pltpu.make_async_copy(dtab_hbm.at[didx[b]], drows[b],
                                  gsem[b]).wait()

        def compute(b):
            ro, dro, oro = rows[b], drows[b], orows[b]

            @plsc.parallel_loop(0, B, unroll=4)
            def _edges(ei):
                ps = []
                for kr in range(drep // 16):
                    a = ro[ei, pl.ds(dh + 16 * kr, 16)] + \
                        dro[ei, pl.ds(16 * kr, 16)]
                    a = jnp.where(a >= 0, a, 0.2 * a)
                    p = jnp.exp(a)
                    ps.append(p)
                    oro[ei, pl.ds(dh + 16 * kr, 16)] = p
                for kk in range(dh // 16):
                    oro[ei, pl.ds(16 * kk, 16)] = (
                        ro[ei, pl.ds(16 * kk, 16)] * ps[kk % len(ps)])

        fire(0, 0)

        def pair(jj, carry):
            j0 = jj * 2
            for b in range(2):
                j = j0 + b
                nb = 1 - b

                @pl.when(j + 1 < nchunks)
                def _fire_next():
                    fire(j + 1, nb)

                wait_gather(b)
                compute(b)
                pltpu.sync_copy(orows[b], accum.at[didx[b]], add=True)
            return carry

        lax.fori_loop(0, nchunks // 2, pair, 0)
        plsc.subcore_barrier()

        def dump(c0):
            pltpu.sync_copy(accum.at[pl.ds(c0 * CH, CH)], zbuf)
            pltpu.sync_copy(zbuf, out_hbm.at[cid, pl.ds(c0 * CH, CH)])

        for t in range(cpt):
            dump(sid * cpt + t)
        if rem:
            @pl.when(sid < rem)
            def _dump_extra():
                dump(cpt * ns + sid)

    return k


_sc_layer1 = _make_sc_edge_kernel(_N, _E, 64, 64)
_sc_layer2 = _make_sc_edge_kernel(_N, _E, 128, 16)


def _block_att_rep(att, heads, ch):
    a = att.reshape(heads, ch)
    eye_h = jnp.eye(heads, dtype=jnp.float32)
    m = a[:, :, None, None] * eye_h[:, None, :, None]
    m = jnp.broadcast_to(m, (heads, ch, heads, ch))
    return m.reshape(heads * ch, heads * ch)


def kernel(x, edge_index, W1, att_src1, att_dst1, b1, W2, att_src2,
           att_dst2, b2):
    f32 = jnp.float32
    src = edge_index[0]
    dst = edge_index[1]

    asrc1 = _block_att_rep(att_src1, 8, 8)
    adst1 = _block_att_rep(att_dst1, 8, 8)
    G1 = jnp.concatenate([jnp.eye(64, dtype=f32), asrc1], axis=1)
    Gd1 = adst1

    as2 = jnp.tile(att_src2.reshape(128, 1), (1, 16))
    ad2 = jnp.tile(att_dst2.reshape(128, 1), (1, 16))
    G2 = jnp.concatenate([jnp.eye(128, dtype=f32), as2], axis=1)
    Gd2 = ad2

    S1 = np.zeros((128, 64), np.float32)
    S1[:64, :64] = np.eye(64)
    R1 = np.zeros((128, 64), np.float32)
    R1[64:, :64] = np.eye(64)
    S2 = np.zeros((144, 128), np.float32)
    S2[:128, :128] = np.eye(128)
    R2 = np.zeros((144, 128), np.float32)
    R2[128, :] = 1.0
    S1, R1, S2, R2 = map(jnp.asarray, (S1, R1, S2, R2))

    b1r = b1.reshape(1, 64)
    b2r = b2.reshape(1, 128)

    RB = 1000
    G = _N // RB

    src_tab, dst_tab = pl.pallas_call(
        _tc1_body,
        grid=(G,),
        in_specs=[
            pl.BlockSpec((RB, 128), lambda i: (i, 0)),
            pl.BlockSpec((128, 64), lambda i: (0, 0)),
            pl.BlockSpec((64, 128), lambda i: (0, 0)),
            pl.BlockSpec((64, 64), lambda i: (0, 0)),
        ],
        out_specs=[
            pl.BlockSpec((RB, 128), lambda i: (i, 0)),
            pl.BlockSpec((RB, 64), lambda i: (i, 0)),
        ],
        out_shape=[
            jax.ShapeDtypeStruct((_N, 128), f32),
            jax.ShapeDtypeStruct((_N, 64), f32),
        ],
    )(x, W1, G1, Gd1)

    acc1 = _sc_layer1(src, dst, src_tab, dst_tab)

    src_tab2, dst_tab2 = pl.pallas_call(
        _tc2_body,
        grid=(G,),
        in_specs=[
            pl.BlockSpec((RB, 128), lambda i: (i, 0)),
            pl.BlockSpec((RB, 128), lambda i: (i, 0)),
            pl.BlockSpec((128, 64), lambda i: (0, 0)),
            pl.BlockSpec((128, 64), lambda i: (0, 0)),
            pl.BlockSpec((1, 64), lambda i: (0, 0)),
            pl.BlockSpec((64, 128), lambda i: (0, 0)),
            pl.BlockSpec((128, 144), lambda i: (0, 0)),
            pl.BlockSpec((128, 16), lambda i: (0, 0)),
        ],
        out_specs=[
            pl.BlockSpec((RB, 144), lambda i: (i, 0)),
            pl.BlockSpec((RB, 16), lambda i: (i, 0)),
        ],
        out_shape=[
            jax.ShapeDtypeStruct((_N, 144), f32),
            jax.ShapeDtypeStruct((_N, 16), f32),
        ],
    )(acc1[0], acc1[1], S1, R1, b1r, W2, G2, Gd2)

    acc2 = _sc_layer2(src, dst, src_tab2, dst_tab2)

    out = pl.pallas_call(
        _tc3_body,
        grid=(G,),
        in_specs=[
            pl.BlockSpec((RB, 144), lambda i: (i, 0)),
            pl.BlockSpec((RB, 144), lambda i: (i, 0)),
            pl.BlockSpec((144, 128), lambda i: (0, 0)),
            pl.BlockSpec((144, 128), lambda i: (0, 0)),
            pl.BlockSpec((1, 128), lambda i: (0, 0)),
        ],
        out_specs=pl.BlockSpec((RB, 128), lambda i: (i, 0)),
        out_shape=jax.ShapeDtypeStruct((_N, 128), f32),
    )(acc2[0], acc2[1], S2, R2, b2r)

    return out


# speedup vs baseline: 71.3581x; 1.2245x over previous
import functools
import numpy as np
import jax
import jax.numpy as jnp
from jax import lax
from jax.experimental import pallas as pl
from jax.experimental.pallas import tpu as pltpu
from jax.experimental.pallas import tpu_sc as plsc

_N = 10000
_E = 320000


def _tc1_body(x_ref, w_ref, g_ref, gd_ref, src_ref, dst_ref):
    h = jnp.dot(x_ref[...], w_ref[...], preferred_element_type=jnp.float32)
    src_ref[...] = jnp.dot(h, g_ref[...], preferred_element_type=jnp.float32)
    dst_ref[...] = jnp.dot(h, gd_ref[...], preferred_element_type=jnp.float32)


def _tc2_body(a0_ref, a1_ref, s_ref, r_ref, b_ref, w2_ref, g2_ref, gd2_ref,
              src2_ref, dst2_ref):
    acc = a0_ref[...] + a1_ref[...]
    numer = jnp.dot(acc, s_ref[...], preferred_element_type=jnp.float32)
    denom = jnp.dot(acc, r_ref[...], preferred_element_type=jnp.float32)
    o = numer / (denom + 1e-16) + b_ref[...]
    o = jnp.where(o > 0, o, jnp.exp(o) - 1.0)
    h2 = jnp.dot(o, w2_ref[...], preferred_element_type=jnp.float32)
    src2_ref[...] = jnp.dot(h2, g2_ref[...], preferred_element_type=jnp.float32)
    dst2_ref[...] = jnp.dot(h2, gd2_ref[...], preferred_element_type=jnp.float32)


def _tc3_body(a0_ref, a1_ref, s_ref, r_ref, b_ref, out_ref):
    acc = a0_ref[...] + a1_ref[...]
    numer = jnp.dot(acc, s_ref[...], preferred_element_type=jnp.float32)
    denom = jnp.dot(acc, r_ref[...], preferred_element_type=jnp.float32)
    z = numer / (denom + 1e-16) + b_ref[...]
    m = jnp.max(z, axis=1, keepdims=True)
    out_ref[...] = z - (m + jnp.log(jnp.sum(jnp.exp(z - m), axis=1,
                                            keepdims=True)))


def _make_sc_edge_kernel(n, e, dh, heads, B):
    drep = 16
    row = dh + drep
    info = plsc.get_sparse_core_info()
    nc, ns = info.num_cores, info.num_subcores
    nw = nc * ns
    epw = e // nw
    nchunks = epw // B
    CH = 40
    nch = n // CH
    cpt = nch // ns
    rem = nch - cpt * ns
    assert epw * nw == e and nchunks * B == epw and nch * CH == n
    mesh = plsc.VectorSubcoreMesh(core_axis_name="c", subcore_axis_name="s")

    @functools.partial(
        pl.kernel,
        out_type=jax.ShapeDtypeStruct((nc, n, row), jnp.float32),
        mesh=mesh,
        compiler_params=pltpu.CompilerParams(use_tc_tiling_on_sc=False),
        scratch_types=[
            pltpu.VMEM((B,), jnp.int32),
            pltpu.VMEM((B,), jnp.int32),
            pltpu.VMEM((B,), jnp.int32),
            pltpu.VMEM((B,), jnp.int32),
            pltpu.VMEM((B, row), jnp.float32),
            pltpu.VMEM((B, row), jnp.float32),
            pltpu.VMEM((B, drep), jnp.float32),
            pltpu.VMEM((B, drep), jnp.float32),
            pltpu.VMEM((B, row), jnp.float32),
            pltpu.VMEM((B, row), jnp.float32),
            pltpu.VMEM((CH, row), jnp.float32),
            pltpu.VMEM_SHARED((n, row), jnp.float32),
            pltpu.SemaphoreType.DMA,
            pltpu.SemaphoreType.DMA,
        ],
    )
    def k(src_hbm, dst_hbm, stab_hbm, dtab_hbm, out_hbm,
          sidx0, sidx1, didx0, didx1,
          rows0, rows1, drows0, drows1, orows0, orows1,
          zbuf, accum, gsem0, gsem1):
        sidx = (sidx0, sidx1)
        didx = (didx0, didx1)
        rows = (rows0, rows1)
        drows = (drows0, drows1)
        orows = (orows0, orows1)
        gsem = (gsem0, gsem1)
        cid = lax.axis_index("c")
        sid = lax.axis_index("s")
        wid = sid * nc + cid
        zero = jnp.zeros((16,), jnp.float32)

        def zrow(i, carry):
            for t in range(row // 16):
                zbuf[i, pl.ds(16 * t, 16)] = zero
            return carry

        lax.fori_loop(0, CH, zrow, 0)
        for t in range(cpt):
            c0 = sid * cpt + t
            pltpu.sync_copy(zbuf, accum.at[pl.ds(c0 * CH, CH)])
        if rem:
            @pl.when(sid < rem)
            def _zero_extra():
                c0 = cpt * ns + sid
                pltpu.sync_copy(zbuf, accum.at[pl.ds(c0 * CH, CH)])
        plsc.subcore_barrier()

        def fire(j, b):
            base = wid * epw + j * B
            pltpu.sync_copy(src_hbm.at[pl.ds(base, B)], sidx[b])
            pltpu.sync_copy(dst_hbm.at[pl.ds(base, B)], didx[b])
            pltpu.async_copy(stab_hbm.at[sidx[b]], rows[b], gsem[b])
            pltpu.async_copy(dtab_hbm.at[didx[b]], drows[b], gsem[b])

        def wait_gather(b):
            pltpu.make_async_copy(stab_hbm.at[sidx[b]], rows[b],
                                  gsem[b]).wait()
            pltpu.make_async_copy(dtab_hbm.at[didx[b]], drows[b],
                                  gsem[b]).wait()

        def compute(b):
            ro, dro, oro = rows[b], drows[b], orows[b]

            @plsc.parallel_loop(0, B, unroll=4)
            def _edges(ei):
                vas = ro[ei, pl.ds(dh, 16)]
                vad = dro[ei, pl.ds(0, 16)]
                a = vas + vad
                a = jnp.where(a >= 0, a, 0.2 * a)
                p = jnp.exp(a)
                oro[ei, pl.ds(dh, 16)] = p
                for kk in range(dh // 16):
                    if heads == 1:
                        pidx = jnp.zeros((16,), jnp.int32)
                    else:
                        pidx = 2 * kk + lax.shift_right_logical(
                            lax.iota(jnp.int32, 16), 3)
                    sv = p.at[pidx].get(mode='promise_in_bounds')
                    oro[ei, pl.ds(16 * kk, 16)] = (
                        ro[ei, pl.ds(16 * kk, 16)] * sv)

        fire(0, 0)

        def pair(jj, carry):
            j0 = jj * 2
            for b in range(2):
                j = j0 + b
                nb = 1 - b

                @pl.when(j < nchunks)
                def _section():
                    @pl.when(j + 1 < nchunks)
                    def _fire_next():
                        fire(j + 1, nb)

                    wait_gather(b)
                    compute(b)
                    pltpu.sync_copy(orows[b], accum.at[didx[b]], add=True)
            return carry

        lax.fori_loop(0, (nchunks + 1) // 2, pair, 0)
        plsc.subcore_barrier()

        def dump(c0):
            pltpu.sync_copy(accum.at[pl.ds(c0 * CH, CH)], zbuf)
            pltpu.sync_copy(zbuf, out_hbm.at[cid, pl.ds(c0 * CH, CH)])

        for t in range(cpt):
            dump(sid * cpt + t)
        if rem:
            @pl.when(sid < rem)
            def _dump_extra():
                dump(cpt * ns + sid)

    return k


_sc_layer1 = _make_sc_edge_kernel(_N, _E, 64, 8, 80)
_sc_layer2 = _make_sc_edge_kernel(_N, _E, 128, 1, 40)


def _block_att(att, heads, ch):
    a = att.reshape(heads, ch)
    eye_h = jnp.eye(heads, dtype=jnp.float32)
    return (a[:, :, None] * eye_h[:, None, :]).reshape(heads * ch, heads)


def kernel(x, edge_index, W1, att_src1, att_dst1, b1, W2, att_src2,
           att_dst2, b2):
    f32 = jnp.float32
    src = edge_index[0]
    dst = edge_index[1]

    asrc1 = _block_att(att_src1, 8, 8)
    adst1 = _block_att(att_dst1, 8, 8)
    z64_8 = jnp.zeros((64, 8), f32)
    G1 = jnp.concatenate([jnp.eye(64, dtype=f32), asrc1, z64_8], axis=1)
    Gd1 = jnp.concatenate([adst1, z64_8], axis=1)

    z128_15 = jnp.zeros((128, 15), f32)
    G2 = jnp.concatenate([jnp.eye(128, dtype=f32), att_src2.reshape(128, 1),
                          z128_15], axis=1)
    Gd2 = jnp.concatenate([att_dst2.reshape(128, 1), z128_15], axis=1)

    S1 = np.zeros((80, 64), np.float32)
    S1[:64, :64] = np.eye(64)
    R1 = np.zeros((80, 64), np.float32)
    for h in range(8):
        R1[64 + h, h * 8:(h + 1) * 8] = 1.0
    S2 = np.zeros((144, 128), np.float32)
    S2[:128, :128] = np.eye(128)
    R2 = np.zeros((144, 128), np.float32)
    R2[128, :] = 1.0
    S1, R1, S2, R2 = map(jnp.asarray, (S1, R1, S2, R2))

    b1r = b1.reshape(1, 64)
    b2r = b2.reshape(1, 128)

    RB = 1000
    G = _N // RB

    src_tab, dst_tab = pl.pallas_call(
        _tc1_body,
        grid=(G,),
        in_specs=[
            pl.BlockSpec((RB, 128), lambda i: (i, 0)),
            pl.BlockSpec((128, 64), lambda i: (0, 0)),
            pl.BlockSpec((64, 80), lambda i: (0, 0)),
            pl.BlockSpec((64, 16), lambda i: (0, 0)),
        ],
        out_specs=[
            pl.BlockSpec((RB, 80), lambda i: (i, 0)),
            pl.BlockSpec((RB, 16), lambda i: (i, 0)),
        ],
        out_shape=[
            jax.ShapeDtypeStruct((_N, 80), f32),
            jax.ShapeDtypeStruct((_N, 16), f32),
        ],
    )(x, W1, G1, Gd1)

    acc1 = _sc_layer1(src, dst, src_tab, dst_tab)

    src_tab2, dst_tab2 = pl.pallas_call(
        _tc2_body,
        grid=(G,),
        in_specs=[
            pl.BlockSpec((RB, 80), lambda i: (i, 0)),
            pl.BlockSpec((RB, 80), lambda i: (i, 0)),
            pl.BlockSpec((80, 64), lambda i: (0, 0)),
            pl.BlockSpec((80, 64), lambda i: (0, 0)),
            pl.BlockSpec((1, 64), lambda i: (0, 0)),
            pl.BlockSpec((64, 128), lambda i: (0, 0)),
            pl.BlockSpec((128, 144), lambda i: (0, 0)),
            pl.BlockSpec((128, 16), lambda i: (0, 0)),
        ],
        out_specs=[
            pl.BlockSpec((RB, 144), lambda i: (i, 0)),
            pl.BlockSpec((RB, 16), lambda i: (i, 0)),
        ],
        out_shape=[
            jax.ShapeDtypeStruct((_N, 144), f32),
            jax.ShapeDtypeStruct((_N, 16), f32),
        ],
    )(acc1[0], acc1[1], S1, R1, b1r, W2, G2, Gd2)

    acc2 = _sc_layer2(src, dst, src_tab2, dst_tab2)

    out = pl.pallas_call(
        _tc3_body,
        grid=(G,),
        in_specs=[
            pl.BlockSpec((RB, 144), lambda i: (i, 0)),
            pl.BlockSpec((RB, 144), lambda i: (i, 0)),
            pl.BlockSpec((144, 128), lambda i: (0, 0)),
            pl.BlockSpec((144, 128), lambda i: (0, 0)),
            pl.BlockSpec((1, 128), lambda i: (0, 0)),
        ],
        out_specs=pl.BlockSpec((RB, 128), lambda i: (i, 0)),
        out_shape=jax.ShapeDtypeStruct((_N, 128), f32),
    )(acc2[0], acc2[1], S2, R2, b2r)

    return out
